# baseline (speedup 1.0000x reference)
import functools

import jax
import jax.numpy as jnp
from jax import lax
from jax.experimental import pallas as pl
from jax.experimental.pallas import tpu as pltpu
from jax.experimental.pallas import tpu_sc as plsc

_B = 16384
_D = 32
_W = 128
_NC = 2
_NS = 16
_NW = _NC * _NS
_N = _B // _NW
_CHUNK = 128
_NCHUNK = 4 * (_N // _CHUNK)
_SIZES = (1000000, 100000, 1000, 100000)


def _sc_body(idx_hbm, item_hbm, cat_hbm, cup_hbm, user_hbm, out_hbm,
             idx_v, sup_v, off_v, slab_v, out_v, sem0, sem1):
    wid = lax.axis_index("s") * _NC + lax.axis_index("c")
    base = wid * _N
    tables = (item_hbm, cat_hbm, cup_hbm, user_hbm)
    sems = (sem0, sem1)

    pltpu.sync_copy(idx_hbm.at[wid], idx_v)

    for cj in range(_NCHUNK):
        size = _SIZES[cj // 4]
        def _mod_body(i, _, cj=cj, size=size):
            sl = pl.ds(i * 16, 16)
            m = lax.rem(idx_v[cj, sl], size)
            sup_v[cj, sl] = lax.shift_right_logical(m, 2)
            off_v[cj, sl] = lax.shift_left(lax.bitwise_and(m, 3), 5)
            return 0
        lax.fori_loop(0, _CHUNK // 16, _mod_body, 0)

    def _fire(cj):
        pltpu.async_copy(
            tables[cj // 4].at[sup_v.at[cj]],
            slab_v.at[cj % 2],
            sems[cj % 2])

    def _drain(cj):
        pltpu.make_async_copy(
            tables[0].at[pl.ds(0, _CHUNK)], slab_v.at[cj % 2],
            sems[cj % 2]).wait()

    def _extract(cj):
        c = cj // 4
        slab = slab_v.at[cj % 2]
        lane = lax.iota(jnp.int32, 16)
        def _rows(i, _, cj=cj, c=c, slab=slab, lane=lane):
            rows = i * 16 + lane
            src_col = off_v[cj, pl.ds(i * 16, 16)]
            dst_rows = (cj % 4) * _CHUNK + rows
            for w in range(_D):
                v = plsc.load_gather(slab, [rows, src_col + w])
                plsc.store_scatter(out_v, [dst_rows, lane * 0 + (c * _D + w)], v)
            return 0
        lax.fori_loop(0, _CHUNK // 16, _rows, 0)

    _fire(0)
    for cj in range(_NCHUNK):
        if cj + 1 < _NCHUNK:
            _fire(cj + 1)
        _drain(cj)
        _extract(cj)

    pltpu.sync_copy(out_v, out_hbm.at[pl.ds(base, _N)])


@jax.jit
def kernel(batch_input, item_table, category_table, cup_size_table,
           user_table):
    idx = batch_input.astype(jnp.int32)
    idx = (idx.reshape(_NW, _N // _CHUNK, _CHUNK, 4)
              .transpose(0, 3, 1, 2).reshape(_NW, _NCHUNK, _CHUNK))

    tabs = [t.reshape(-1, _W) for t in
            (item_table, category_table, cup_size_table, user_table)]

    mesh = plsc.VectorSubcoreMesh(core_axis_name="c", subcore_axis_name="s")
    run = functools.partial(
        pl.kernel,
        mesh=mesh,
        compiler_params=pltpu.CompilerParams(needs_layout_passes=False),
        out_type=jax.ShapeDtypeStruct((_B, 4 * _D), jnp.float32),
        scratch_types=[
            pltpu.VMEM((_NCHUNK, _CHUNK), jnp.int32),
            pltpu.VMEM((_NCHUNK, _CHUNK), jnp.int32),
            pltpu.VMEM((_NCHUNK, _CHUNK), jnp.int32),
            pltpu.VMEM((2, _CHUNK, _W), jnp.float32),
            pltpu.VMEM((_N, 4 * _D), jnp.float32),
            pltpu.SemaphoreType.DMA,
            pltpu.SemaphoreType.DMA,
        ],
    )(_sc_body)
    return run(idx, *tabs)


# speedup vs baseline: 1.1346x; 1.1346x over previous
import functools

import jax
import jax.numpy as jnp
from jax import lax
from jax.experimental import pallas as pl
from jax.experimental.pallas import tpu as pltpu
from jax.experimental.pallas import tpu_sc as plsc

_B = 16384
_D = 32
_NC = 2
_NS = 16
_NW = _NC * _NS
_N = _B // _NW
_CHUNK = 128
_NCHUNK = _N // _CHUNK
_SIZES = (1000000, 100000, 1000, 100000)


def _sc_body(idx_hbm, item_hbm, cat_hbm, cup_hbm, user_hbm, out_hbm,
             idx_v, rows_v, sem):
    wid = lax.axis_index("s") * _NC + lax.axis_index("c")
    base = wid * _N

    pltpu.sync_copy(idx_hbm.at[wid], idx_v)

    for c, size in enumerate(_SIZES):
        for j in range(_NCHUNK):
            def _mod_body(i, _, c=c, j=j, size=size):
                sl = pl.ds(i * 16, 16)
                idx_v[c, j, sl] = lax.rem(idx_v[c, j, sl], size)
                return 0
            lax.fori_loop(0, _CHUNK // 16, _mod_body, 0)

    tables = (item_hbm, cat_hbm, cup_hbm, user_hbm)
    for c, tab in enumerate(tables):
        for j in range(_NCHUNK):
            pltpu.async_copy(
                tab.at[idx_v.at[c, j]],
                rows_v.at[c, pl.ds(j * _CHUNK, _CHUNK)],
                sem)
    for c in range(4):
        pltpu.make_async_copy(
            tables[c].at[pl.ds(0, _N)], rows_v.at[c], sem).wait()

    for c in range(4):
        pltpu.sync_copy(rows_v.at[c],
                        out_hbm.at[pl.ds(base, _N), pl.ds(c * _D, _D)])


@jax.jit
def kernel(batch_input, item_table, category_table, cup_size_table,
           user_table):
    idx = batch_input.astype(jnp.int32)
    idx = idx.reshape(_NW, _NCHUNK, _CHUNK, 4).transpose(0, 3, 1, 2)

    mesh = plsc.VectorSubcoreMesh(core_axis_name="c", subcore_axis_name="s")
    run = functools.partial(
        pl.kernel,
        mesh=mesh,
        compiler_params=pltpu.CompilerParams(use_tc_tiling_on_sc=False),
        out_type=jax.ShapeDtypeStruct((_B, 4 * _D), jnp.float32),
        scratch_types=[
            pltpu.VMEM((4, _NCHUNK, _CHUNK), jnp.int32),
            pltpu.VMEM((4, _N, _D), jnp.float32),
            pltpu.SemaphoreType.DMA,
        ],
    )(_sc_body)
    return run(idx, item_table, category_table, cup_size_table, user_table)
